# baseline (device time: 10684 ns/iter reference)
import jax
import jax.numpy as jnp
from jax import lax
from jax.experimental import pallas as pl
from jax.experimental.pallas import tpu as pltpu

BLOCK_M = 256


def kernel(x, dy, gamma):
    del gamma
    m, d = x.shape
    n_chunks = m // BLOCK_M

    def body(x_hbm, dy_hbm, out_hbm, xbuf, dybuf, load_sems,
             comm_ref, send_sem, recv_sem, out_sem):
        my_x = lax.axis_index("x")
        my_y = lax.axis_index("y")
        my_z = lax.axis_index("z")
        peer = (1 - my_x, my_y, my_z)

        barrier_sem = pltpu.get_barrier_semaphore()
        pl.semaphore_signal(
            barrier_sem, inc=1, device_id=peer,
            device_id_type=pl.DeviceIdType.MESH,
        )

        def chunk_copies(i, slot):
            rows = pl.ds(i * BLOCK_M, BLOCK_M)
            return (
                pltpu.make_async_copy(
                    x_hbm.at[rows, :], xbuf.at[slot], load_sems.at[slot, 0]
                ),
                pltpu.make_async_copy(
                    dy_hbm.at[rows, :], dybuf.at[slot], load_sems.at[slot, 1]
                ),
            )

        for c in chunk_copies(0, 0):
            c.start()

        acc_g = jnp.zeros((d,), jnp.float32)
        acc_b = jnp.zeros((d,), jnp.float32)
        for i in range(n_chunks):
            slot = i % 2
            if i + 1 < n_chunks:
                for c in chunk_copies(i + 1, (i + 1) % 2):
                    c.start()
            for c in chunk_copies(i, slot):
                c.wait()
            xv = xbuf[slot]
            dyv = dybuf[slot]
            mu = jnp.mean(xv, axis=1, keepdims=True)
            diff = xv - mu
            var = jnp.mean(diff * diff, axis=1, keepdims=True)
            xhat = diff * lax.rsqrt(var + 1e-5)
            acc_g = acc_g + jnp.sum(dyv * xhat, axis=0)
            acc_b = acc_b + jnp.sum(dyv, axis=0)

        comm_ref[0, 0, :] = acc_g
        comm_ref[0, 1, :] = acc_b

        pl.semaphore_wait(barrier_sem, 1)
        rdma = pltpu.make_async_remote_copy(
            src_ref=comm_ref.at[0],
            dst_ref=comm_ref.at[1],
            send_sem=send_sem,
            recv_sem=recv_sem,
            device_id=peer,
            device_id_type=pl.DeviceIdType.MESH,
        )
        rdma.start()
        rdma.wait_recv()
        comm_ref[2, :, :] = comm_ref[0] + comm_ref[1]
        out_copy = pltpu.make_async_copy(comm_ref.at[2], out_hbm, out_sem)
        out_copy.start()
        out_copy.wait()
        rdma.wait_send()

    return pl.pallas_call(
        body,
        out_shape=jax.ShapeDtypeStruct((2, d), jnp.float32),
        in_specs=[
            pl.BlockSpec(memory_space=pltpu.MemorySpace.HBM),
            pl.BlockSpec(memory_space=pltpu.MemorySpace.HBM),
        ],
        out_specs=pl.BlockSpec(memory_space=pltpu.MemorySpace.HBM),
        scratch_shapes=[
            pltpu.VMEM((2, BLOCK_M, d), jnp.float32),
            pltpu.VMEM((2, BLOCK_M, d), jnp.float32),
            pltpu.SemaphoreType.DMA((2, 2)),
            pltpu.VMEM((3, 2, d), jnp.float32),
            pltpu.SemaphoreType.DMA,
            pltpu.SemaphoreType.DMA,
            pltpu.SemaphoreType.DMA,
        ],
        compiler_params=pltpu.CompilerParams(collective_id=0),
    )(x, dy)


# device time: 8297 ns/iter; 1.2877x vs baseline; 1.2877x over previous
import jax
import jax.numpy as jnp
from jax import lax
from jax.experimental import pallas as pl
from jax.experimental.pallas import tpu as pltpu

CHUNKS = ((0, 640), (640, 384))
MAX_ROWS = max(sz for _, sz in CHUNKS)


def kernel(x, dy, gamma):
    del gamma
    m, d = x.shape
    n_chunks = len(CHUNKS)

    def body(x_hbm, dy_hbm, out_hbm, xbuf, dybuf, load_sems,
             comm_ref, send_sem, recv_sem, out_sem):
        my_x = lax.axis_index("x")
        my_y = lax.axis_index("y")
        my_z = lax.axis_index("z")
        peer = (1 - my_x, my_y, my_z)

        barrier_sem = pltpu.get_barrier_semaphore()
        pl.semaphore_signal(
            barrier_sem, inc=1, device_id=peer,
            device_id_type=pl.DeviceIdType.MESH,
        )

        def chunk_copies(ci, slot):
            off, sz = CHUNKS[ci]
            rows = pl.ds(off, sz)
            buf_rows = pl.ds(0, sz)
            return (
                pltpu.make_async_copy(
                    x_hbm.at[rows, :], xbuf.at[slot, buf_rows, :],
                    load_sems.at[slot, 0],
                ),
                pltpu.make_async_copy(
                    dy_hbm.at[rows, :], dybuf.at[slot, buf_rows, :],
                    load_sems.at[slot, 1],
                ),
            )

        for c in chunk_copies(0, 0):
            c.start()

        acc_g = jnp.zeros((d,), jnp.float32)
        acc_b = jnp.zeros((d,), jnp.float32)
        for i in range(n_chunks):
            slot = i % 2
            if i + 1 < n_chunks:
                for c in chunk_copies(i + 1, (i + 1) % 2):
                    c.start()
            for c in chunk_copies(i, slot):
                c.wait()
            sz = CHUNKS[i][1]
            xv = xbuf[slot, 0:sz, :]
            dyv = dybuf[slot, 0:sz, :]
            mu = jnp.mean(xv, axis=1, keepdims=True)
            diff = xv - mu
            var = jnp.mean(diff * diff, axis=1, keepdims=True)
            xhat = diff * lax.rsqrt(var + 1e-5)
            acc_g = acc_g + jnp.sum(dyv * xhat, axis=0)
            acc_b = acc_b + jnp.sum(dyv, axis=0)

        comm_ref[0, 0, :] = acc_g
        comm_ref[0, 1, :] = acc_b

        pl.semaphore_wait(barrier_sem, 1)
        rdma = pltpu.make_async_remote_copy(
            src_ref=comm_ref.at[0],
            dst_ref=comm_ref.at[1],
            send_sem=send_sem,
            recv_sem=recv_sem,
            device_id=peer,
            device_id_type=pl.DeviceIdType.MESH,
        )
        rdma.start()
        rdma.wait_recv()
        comm_ref[2, :, :] = comm_ref[0] + comm_ref[1]
        out_copy = pltpu.make_async_copy(comm_ref.at[2], out_hbm, out_sem)
        out_copy.start()
        out_copy.wait()
        rdma.wait_send()

    return pl.pallas_call(
        body,
        out_shape=jax.ShapeDtypeStruct((2, d), jnp.float32),
        in_specs=[
            pl.BlockSpec(memory_space=pltpu.MemorySpace.HBM),
            pl.BlockSpec(memory_space=pltpu.MemorySpace.HBM),
        ],
        out_specs=pl.BlockSpec(memory_space=pltpu.MemorySpace.HBM),
        scratch_shapes=[
            pltpu.VMEM((2, MAX_ROWS, d), jnp.float32),
            pltpu.VMEM((2, MAX_ROWS, d), jnp.float32),
            pltpu.SemaphoreType.DMA((2, 2)),
            pltpu.VMEM((3, 2, d), jnp.float32),
            pltpu.SemaphoreType.DMA,
            pltpu.SemaphoreType.DMA,
            pltpu.SemaphoreType.DMA,
        ],
        compiler_params=pltpu.CompilerParams(collective_id=0),
    )(
        pltpu.with_memory_space_constraint(x, pltpu.MemorySpace.HBM),
        pltpu.with_memory_space_constraint(dy, pltpu.MemorySpace.HBM),
    )
